# baseline (speedup 1.0000x reference)
import jax
import jax.numpy as jnp
import numpy as np
from jax import lax
from jax.experimental import pallas as pl
from jax.experimental.pallas import tpu as pltpu
from jax.experimental.pallas import tpu_sc as plsc

M = 16384
S = 5
NUM_CORES = 2
NUM_SUBCORES = 16
NUM_TILES = NUM_CORES * NUM_SUBCORES
CHUNK = M // NUM_TILES
CCHUNK = M // NUM_SUBCORES
CVPC = CCHUNK // 16
BITS_PER_TILE = 2 * S * CHUNK
MARGIN = 1.0

_mesh = plsc.VectorSubcoreMesh(core_axis_name="c", subcore_axis_name="s")


def _tf_hash(k1, k2, c1, c2):
    k1 = np.uint32(k1)
    k2 = np.uint32(k2)
    ks = [k1, k2, np.uint32(k1 ^ k2 ^ np.uint32(0x1BD11BDA))]
    rot = [(13, 15, 26, 6), (17, 29, 16, 24)]
    x0 = (np.asarray(c1, np.uint32) + k1).astype(np.uint32)
    x1 = (np.asarray(c2, np.uint32) + k2).astype(np.uint32)
    for g in range(5):
        for r in rot[g % 2]:
            x0 = (x0 + x1).astype(np.uint32)
            x1 = ((x1 << np.uint32(r)) | (x1 >> np.uint32(32 - r))).astype(np.uint32)
            x1 = (x1 ^ x0).astype(np.uint32)
        x0 = (x0 + ks[(g + 1) % 3]).astype(np.uint32)
        x1 = (x1 + ks[(g + 2) % 3] + np.uint32(g + 1)).astype(np.uint32)
    return x0, x1


def _randint_bits_np():
    base = np.array([0, 42], dtype=np.uint32)
    s0, s1 = _tf_hash(base[0], base[1],
                      np.zeros(2, np.uint32), np.arange(2, dtype=np.uint32))
    n = M * S
    zeros = np.zeros(n, np.uint32)
    iota = np.arange(n, dtype=np.uint32)
    hx0, hx1 = _tf_hash(s0[0], s1[0], zeros, iota)
    lx0, lx1 = _tf_hash(s0[1], s1[1], zeros, iota)
    hb = (hx0 ^ hx1).reshape(M, S).T.reshape(S, NUM_TILES, CHUNK)
    lb = (lx0 ^ lx1).reshape(M, S).T.reshape(S, NUM_TILES, CHUNK)
    bits = np.stack([hb, lb], 0)
    return np.ascontiguousarray(bits.transpose(2, 1, 0, 3)).reshape(-1)


_BITS = _randint_bits_np()


def _body(scores_hbm, target_hbm, bits_hbm, parts_hbm, p_hbm, bounce_hbm,
          tgt_v, sc_v, idx_v, cshared, negv, posv, bitv, pv, accv,
          sem0, sem1, sem2):
    cid = lax.axis_index("c")
    sid = lax.axis_index("s")
    wid = cid * NUM_SUBCORES + sid
    cbase = sid * CCHUNK
    rbase = wid * CHUNK

    scope = jax.named_scope
    tgt_d = pltpu.async_copy(target_hbm, tgt_v, sem0)
    bits_d = pltpu.async_copy(
        bits_hbm.at[pl.ds(wid * BITS_PER_TILE, BITS_PER_TILE)], bitv, sem1)
    sc_d = [pltpu.async_copy(scores_hbm.at[pl.ds(cbase + q * 128, 128)],
                             sc_v.at[q], sem2)
            for q in range(8)]
    with scope("ph_tgtwait"):
        tgt_d.wait()

    zero = jnp.zeros((16,), jnp.int32)
    _ps = scope("ph_prefix"); _ps.__enter__()
    acc_pre = zero
    acc_tot = zero
    for c in range(NUM_SUBCORES):
        s_c = zero
        for k in range(CVPC):
            s_c = s_c + tgt_v[pl.ds(c * CCHUNK + k * 16, 16)]
        m = (c < sid).astype(jnp.int32)
        acc_pre = acc_pre + s_c * m
        acc_tot = acc_tot + s_c
    pos_before = jnp.sum(acc_pre)
    p_total = jnp.sum(acc_tot)
    _ps.__exit__(None, None, None)
    _rs = scope("ph_rank"); _rs.__enter__()

    iot = lax.iota(jnp.int32, 16)
    carry = jnp.int32(0)
    for k in range(CVPC):
        t = tgt_v[pl.ds(cbase + k * 16, 16)]
        csum = plsc.cumsum(t)
        excl = csum - t
        prank = pos_before + carry + excl
        gidx = cbase + k * 16 + iot
        dest = jnp.where(t == 1, prank, M + gidx - prank)
        idx_v[k // 8, pl.ds((k % 8) * 16, 16)] = dest
        carry = carry + csum[15]

    _rs.__exit__(None, None, None)
    with scope("ph_scwait"):
        for d in sc_d:
            d.wait()
    scat_d = [pltpu.async_copy(sc_v.at[q], cshared.at[idx_v.at[q]], sem2)
              for q in range(8)]
    with scope("ph_scatter"):
        for d in scat_d:
            d.wait()

    with scope("ph_bar1"):
        plsc.subcore_barrier()

    out_off = cid * 2 * M + sid * (2 * M // NUM_SUBCORES)
    with scope("ph_copyout"):
        pltpu.sync_copy(cshared.at[pl.ds(sid * (2 * M // NUM_SUBCORES),
                                         2 * M // NUM_SUBCORES)],
                        bounce_hbm.at[pl.ds(out_off, 2 * M // NUM_SUBCORES)])

    with scope("ph_bar2"):
        plsc.subcore_barrier()

    neg_copy = pltpu.async_copy(bounce_hbm.at[pl.ds(cid * 2 * M + M, M)],
                                negv, sem0)
    pos_copy = pltpu.async_copy(bounce_hbm.at[pl.ds(cid * 2 * M + rbase, CHUNK)],
                                posv, sem2)

    P = jnp.broadcast_to(p_total, (16,))
    s_i = jnp.maximum(M - P, 1)
    s_u = plsc.bitcast(s_i, jnp.uint32)
    s_f = s_i.astype(jnp.float32)
    rcp = 1.0 / s_f
    m1 = lax.rem(jnp.full((16,), 65536, jnp.uint32), s_u)
    m2 = lax.rem(m1 * m1, s_u)
    a3 = lax.rem(m2 * m1, s_u)
    off = s_i * (2 + lax.div(1024 + s_i - 1, s_i))
    lim = jnp.full((16,), 2.0e9, jnp.float32)
    big = jnp.full((16,), 4294967296.0, jnp.float32)
    mask16 = jnp.full((16,), 0xFFFF, jnp.uint32)
    is_one = s_i == 1

    with scope("ph_stage"):
        bits_d.wait()
        pos_copy.wait()
        neg_copy.wait()

    def body(i, acc):
        sc = i // (CHUNK // 16)
        k = i - sc * (CHUNK // 16)
        hb = bitv[pl.ds(sc * 2 * CHUNK + k * 16, 16)]
        lb = bitv[pl.ds(sc * 2 * CHUNK + CHUNK + k * 16, 16)]
        h1 = lax.shift_right_logical(hb, jnp.uint32(16))
        h0 = hb & mask16
        l1 = lax.shift_right_logical(lb, jnp.uint32(16))
        l0 = lb & mask16
        v = h1 * a3 + h0 * m2 + l1 * m1 + l0
        vi = plsc.bitcast(v, jnp.int32)
        vf = vi.astype(jnp.float32)
        vf = jnp.where(vi < 0, vf + big, vf)
        q1 = jnp.minimum(vf * rcp, lim).astype(jnp.int32)
        r1u = v - plsc.bitcast(q1, jnp.uint32) * s_u
        r1 = plsc.bitcast(r1u, jnp.int32) + off
        q2 = (r1.astype(jnp.float32) * rcp).astype(jnp.int32)
        r2 = r1 - q2 * s_i
        r2 = jnp.where(r2 < 0, r2 + s_i, r2)
        r2 = jnp.where(r2 >= s_i, r2 - s_i, r2)
        samp = jnp.where(is_one, 0, r2)
        neg = plsc.load_gather(negv, [samp])
        p = posv[pl.ds(k * 16, 16)]
        rowid = rbase + k * 16 + iot
        term = jnp.maximum(MARGIN - p + neg, 0.0)
        return acc + jnp.where(rowid < P, term, 0.0)

    _ls = scope("ph_sample"); _ls.__enter__()
    acc = lax.fori_loop(0, S * (CHUNK // 16), body, jnp.zeros((16,), jnp.float32))
    accv[...] = acc
    _ls.__exit__(None, None, None)
    pltpu.sync_copy(accv, parts_hbm.at[pl.ds(wid * 16, 16)])

    @pl.when(wid == 0)
    def _():
        pv[...] = P
        pltpu.sync_copy(pv, p_hbm)


_fused = pl.kernel(
    _body,
    out_type=[jax.ShapeDtypeStruct((NUM_TILES * 16,), jnp.float32),
              jax.ShapeDtypeStruct((16,), jnp.int32),
              jax.ShapeDtypeStruct((NUM_CORES * 2 * M,), jnp.float32)],
    mesh=_mesh,
    compiler_params=pltpu.CompilerParams(needs_layout_passes=False),
    scratch_types=[
        pltpu.VMEM((M,), jnp.int32),
        pltpu.VMEM((8, 128), jnp.float32),
        pltpu.VMEM((8, 128), jnp.int32),
        pltpu.VMEM_SHARED((2 * M,), jnp.float32),
        pltpu.VMEM((M,), jnp.float32),
        pltpu.VMEM((CHUNK,), jnp.float32),
        pltpu.VMEM((BITS_PER_TILE,), jnp.uint32),
        pltpu.VMEM((16,), jnp.int32),
        pltpu.VMEM((16,), jnp.float32),
        pltpu.SemaphoreType.DMA,
        pltpu.SemaphoreType.DMA,
        pltpu.SemaphoreType.DMA,
    ],
)


def kernel(scores, target):
    bits = jnp.asarray(_BITS)
    parts, pvec, _ = _fused(scores, target, bits)
    P = pvec[0]
    return parts.sum() / (P * S)


# speedup vs baseline: 1.0682x; 1.0682x over previous
import jax
import jax.numpy as jnp
import numpy as np
from jax import lax
from jax.experimental import pallas as pl
from jax.experimental.pallas import tpu as pltpu
from jax.experimental.pallas import tpu_sc as plsc

M = 16384
S = 5
NUM_CORES = 1
NUM_SUBCORES = 16
NUM_TILES = NUM_CORES * NUM_SUBCORES
CHUNK = M // NUM_TILES
CCHUNK = M // NUM_SUBCORES
CVPC = CCHUNK // 16
BITS_PER_TILE = 2 * S * CHUNK
MARGIN = 1.0

_mesh = plsc.VectorSubcoreMesh(core_axis_name="c", subcore_axis_name="s", num_cores=1)


def _tf_hash(k1, k2, c1, c2):
    k1 = np.uint32(k1)
    k2 = np.uint32(k2)
    ks = [k1, k2, np.uint32(k1 ^ k2 ^ np.uint32(0x1BD11BDA))]
    rot = [(13, 15, 26, 6), (17, 29, 16, 24)]
    x0 = (np.asarray(c1, np.uint32) + k1).astype(np.uint32)
    x1 = (np.asarray(c2, np.uint32) + k2).astype(np.uint32)
    for g in range(5):
        for r in rot[g % 2]:
            x0 = (x0 + x1).astype(np.uint32)
            x1 = ((x1 << np.uint32(r)) | (x1 >> np.uint32(32 - r))).astype(np.uint32)
            x1 = (x1 ^ x0).astype(np.uint32)
        x0 = (x0 + ks[(g + 1) % 3]).astype(np.uint32)
        x1 = (x1 + ks[(g + 2) % 3] + np.uint32(g + 1)).astype(np.uint32)
    return x0, x1


def _randint_bits_np():
    base = np.array([0, 42], dtype=np.uint32)
    s0, s1 = _tf_hash(base[0], base[1],
                      np.zeros(2, np.uint32), np.arange(2, dtype=np.uint32))
    n = M * S
    zeros = np.zeros(n, np.uint32)
    iota = np.arange(n, dtype=np.uint32)
    hx0, hx1 = _tf_hash(s0[0], s1[0], zeros, iota)
    lx0, lx1 = _tf_hash(s0[1], s1[1], zeros, iota)
    hb = (hx0 ^ hx1).reshape(M, S).T.reshape(S, NUM_TILES, CHUNK)
    lb = (lx0 ^ lx1).reshape(M, S).T.reshape(S, NUM_TILES, CHUNK)
    bits = np.stack([hb, lb], 0)
    return np.ascontiguousarray(bits.transpose(2, 1, 0, 3)).reshape(-1)


_BITS = _randint_bits_np()


def _body(scores_hbm, target_hbm, bits_hbm, parts_hbm, p_hbm, bounce_hbm,
          tgt_v, sc_v, idx_v, cshared, negv, posv, bitv, pv, accv,
          sem0, sem1, sem2):
    cid = lax.axis_index("c")
    sid = lax.axis_index("s")
    wid = cid * NUM_SUBCORES + sid
    cbase = sid * CCHUNK
    rbase = wid * CHUNK

    scope = jax.named_scope
    tgt_d = pltpu.async_copy(target_hbm, tgt_v, sem0)
    bits_d = pltpu.async_copy(
        bits_hbm.at[pl.ds(wid * BITS_PER_TILE, BITS_PER_TILE)], bitv, sem1)
    sc_d = [pltpu.async_copy(scores_hbm.at[pl.ds(cbase + q * 128, 128)],
                             sc_v.at[q], sem2)
            for q in range(8)]
    with scope("ph_tgtwait"):
        tgt_d.wait()

    zero = jnp.zeros((16,), jnp.int32)
    _ps = scope("ph_prefix"); _ps.__enter__()
    acc_pre = zero
    acc_tot = zero
    for c in range(NUM_SUBCORES):
        s_c = zero
        for k in range(CVPC):
            s_c = s_c + tgt_v[pl.ds(c * CCHUNK + k * 16, 16)]
        m = (c < sid).astype(jnp.int32)
        acc_pre = acc_pre + s_c * m
        acc_tot = acc_tot + s_c
    pos_before = jnp.sum(acc_pre)
    p_total = jnp.sum(acc_tot)
    _ps.__exit__(None, None, None)
    _rs = scope("ph_rank"); _rs.__enter__()

    iot = lax.iota(jnp.int32, 16)
    carry = jnp.int32(0)
    for k in range(CVPC):
        t = tgt_v[pl.ds(cbase + k * 16, 16)]
        csum = plsc.cumsum(t)
        excl = csum - t
        prank = pos_before + carry + excl
        gidx = cbase + k * 16 + iot
        dest = jnp.where(t == 1, prank, M + gidx - prank)
        idx_v[k // 8, pl.ds((k % 8) * 16, 16)] = dest
        carry = carry + csum[15]

    _rs.__exit__(None, None, None)
    with scope("ph_scwait"):
        for d in sc_d:
            d.wait()
    scat_d = [pltpu.async_copy(sc_v.at[q], cshared.at[idx_v.at[q]], sem2)
              for q in range(8)]
    with scope("ph_scatter"):
        for d in scat_d:
            d.wait()

    with scope("ph_bar1"):
        plsc.subcore_barrier()

    out_off = cid * 2 * M + sid * (2 * M // NUM_SUBCORES)
    with scope("ph_copyout"):
        pltpu.sync_copy(cshared.at[pl.ds(sid * (2 * M // NUM_SUBCORES),
                                         2 * M // NUM_SUBCORES)],
                        bounce_hbm.at[pl.ds(out_off, 2 * M // NUM_SUBCORES)])

    with scope("ph_bar2"):
        plsc.subcore_barrier()

    neg_copy = pltpu.async_copy(bounce_hbm.at[pl.ds(cid * 2 * M + M, M)],
                                negv, sem0)
    pos_copy = pltpu.async_copy(bounce_hbm.at[pl.ds(cid * 2 * M + rbase, CHUNK)],
                                posv, sem2)

    P = jnp.broadcast_to(p_total, (16,))
    s_i = jnp.maximum(M - P, 1)
    s_u = plsc.bitcast(s_i, jnp.uint32)
    s_f = s_i.astype(jnp.float32)
    rcp = 1.0 / s_f
    m1 = lax.rem(jnp.full((16,), 65536, jnp.uint32), s_u)
    m2 = lax.rem(m1 * m1, s_u)
    a3 = lax.rem(m2 * m1, s_u)
    off = s_i * (2 + lax.div(1024 + s_i - 1, s_i))
    lim = jnp.full((16,), 2.0e9, jnp.float32)
    big = jnp.full((16,), 4294967296.0, jnp.float32)
    mask16 = jnp.full((16,), 0xFFFF, jnp.uint32)
    is_one = s_i == 1

    with scope("ph_stage"):
        bits_d.wait()
        pos_copy.wait()
        neg_copy.wait()

    def body(i, acc):
        sc = i // (CHUNK // 16)
        k = i - sc * (CHUNK // 16)
        hb = bitv[pl.ds(sc * 2 * CHUNK + k * 16, 16)]
        lb = bitv[pl.ds(sc * 2 * CHUNK + CHUNK + k * 16, 16)]
        h1 = lax.shift_right_logical(hb, jnp.uint32(16))
        h0 = hb & mask16
        l1 = lax.shift_right_logical(lb, jnp.uint32(16))
        l0 = lb & mask16
        v = h1 * a3 + h0 * m2 + l1 * m1 + l0
        vi = plsc.bitcast(v, jnp.int32)
        vf = vi.astype(jnp.float32)
        vf = jnp.where(vi < 0, vf + big, vf)
        q1 = jnp.minimum(vf * rcp, lim).astype(jnp.int32)
        r1u = v - plsc.bitcast(q1, jnp.uint32) * s_u
        r1 = plsc.bitcast(r1u, jnp.int32) + off
        q2 = (r1.astype(jnp.float32) * rcp).astype(jnp.int32)
        r2 = r1 - q2 * s_i
        r2 = jnp.where(r2 < 0, r2 + s_i, r2)
        r2 = jnp.where(r2 >= s_i, r2 - s_i, r2)
        samp = jnp.where(is_one, 0, r2)
        neg = plsc.load_gather(negv, [samp])
        p = posv[pl.ds(k * 16, 16)]
        rowid = rbase + k * 16 + iot
        term = jnp.maximum(MARGIN - p + neg, 0.0)
        return acc + jnp.where(rowid < P, term, 0.0)

    _ls = scope("ph_sample"); _ls.__enter__()
    acc = lax.fori_loop(0, S * (CHUNK // 16), body, jnp.zeros((16,), jnp.float32))
    accv[...] = acc
    _ls.__exit__(None, None, None)
    pltpu.sync_copy(accv, parts_hbm.at[pl.ds(wid * 16, 16)])

    @pl.when(wid == 0)
    def _():
        pv[...] = P
        pltpu.sync_copy(pv, p_hbm)


_fused = pl.kernel(
    _body,
    out_type=[jax.ShapeDtypeStruct((NUM_TILES * 16,), jnp.float32),
              jax.ShapeDtypeStruct((16,), jnp.int32),
              jax.ShapeDtypeStruct((NUM_CORES * 2 * M,), jnp.float32)],
    mesh=_mesh,
    compiler_params=pltpu.CompilerParams(needs_layout_passes=False),
    scratch_types=[
        pltpu.VMEM((M,), jnp.int32),
        pltpu.VMEM((8, 128), jnp.float32),
        pltpu.VMEM((8, 128), jnp.int32),
        pltpu.VMEM_SHARED((2 * M,), jnp.float32),
        pltpu.VMEM((M,), jnp.float32),
        pltpu.VMEM((CHUNK,), jnp.float32),
        pltpu.VMEM((BITS_PER_TILE,), jnp.uint32),
        pltpu.VMEM((16,), jnp.int32),
        pltpu.VMEM((16,), jnp.float32),
        pltpu.SemaphoreType.DMA,
        pltpu.SemaphoreType.DMA,
        pltpu.SemaphoreType.DMA,
    ],
)


def kernel(scores, target):
    bits = jnp.asarray(_BITS)
    parts, pvec, _ = _fused(scores, target, bits)
    P = pvec[0]
    return parts.sum() / (P * S)


# speedup vs baseline: 1.4465x; 1.3541x over previous
import jax
import jax.numpy as jnp
import numpy as np
from jax import lax
from jax.experimental import pallas as pl
from jax.experimental.pallas import tpu as pltpu
from jax.experimental.pallas import tpu_sc as plsc

M = 16384
S = 5
NUM_SUBCORES = 16
CHUNK = M // NUM_SUBCORES
VPC = CHUNK // 16
WPC = CHUNK // 4 // 16
BITS_PER_TILE = 2 * S * CHUNK
MARGIN = 1.0

_mesh = plsc.VectorSubcoreMesh(core_axis_name="c", subcore_axis_name="s",
                               num_cores=1)


def _tf_hash(k1, k2, c1, c2):
    k1 = np.uint32(k1)
    k2 = np.uint32(k2)
    ks = [k1, k2, np.uint32(k1 ^ k2 ^ np.uint32(0x1BD11BDA))]
    rot = [(13, 15, 26, 6), (17, 29, 16, 24)]
    x0 = (np.asarray(c1, np.uint32) + k1).astype(np.uint32)
    x1 = (np.asarray(c2, np.uint32) + k2).astype(np.uint32)
    for g in range(5):
        for r in rot[g % 2]:
            x0 = (x0 + x1).astype(np.uint32)
            x1 = ((x1 << np.uint32(r)) | (x1 >> np.uint32(32 - r))).astype(np.uint32)
            x1 = (x1 ^ x0).astype(np.uint32)
        x0 = (x0 + ks[(g + 1) % 3]).astype(np.uint32)
        x1 = (x1 + ks[(g + 2) % 3] + np.uint32(g + 1)).astype(np.uint32)
    return x0, x1


def _randint_bits_np():
    base = np.array([0, 42], dtype=np.uint32)
    s0, s1 = _tf_hash(base[0], base[1],
                      np.zeros(2, np.uint32), np.arange(2, dtype=np.uint32))
    n = M * S
    zeros = np.zeros(n, np.uint32)
    iota = np.arange(n, dtype=np.uint32)
    hx0, hx1 = _tf_hash(s0[0], s1[0], zeros, iota)
    lx0, lx1 = _tf_hash(s0[1], s1[1], zeros, iota)
    hb = (hx0 ^ hx1).reshape(M, S).T.reshape(S, NUM_SUBCORES, CHUNK)
    lb = (lx0 ^ lx1).reshape(M, S).T.reshape(S, NUM_SUBCORES, CHUNK)
    bits = np.stack([hb, lb], 0)
    return np.ascontiguousarray(bits.transpose(2, 1, 0, 3)).reshape(-1)


_BITS = _randint_bits_np()


def _body(scores_hbm, target_hbm, tgt8_hbm, bits_hbm, loss_hbm, bounce_hbm,
          tgt8_v, tgt_v, sc_v, idx_v, cshared, negv, posv, bitv, redv, pv,
          accv, sem0, sem1, sem2):
    sid = lax.axis_index("s")
    base = sid * CHUNK

    tgt8_d = pltpu.async_copy(tgt8_hbm, tgt8_v, sem0)
    bits_d = pltpu.async_copy(
        bits_hbm.at[pl.ds(sid * BITS_PER_TILE, BITS_PER_TILE)], bitv, sem1)
    tgt_d = pltpu.async_copy(target_hbm.at[pl.ds(base, CHUNK)], tgt_v, sem2)
    sc_d = [pltpu.async_copy(scores_hbm.at[pl.ds(base + q * 128, 128)],
                             sc_v.at[q], sem2)
            for q in range(8)]
    tgt8_d.wait()

    zero = jnp.zeros((16,), jnp.int32)
    sh8 = jnp.int32(8)
    sh16 = jnp.int32(16)
    sh24 = jnp.int32(24)
    b255 = jnp.full((16,), 0xFF, jnp.int32)

    def pref_body(c, carry):
        acc_pre, acc_tot = carry
        w = zero
        for k in range(WPC):
            w = w + tgt8_v[pl.ds(c * (CHUNK // 4) + k * 16, 16)]
        bsum = ((w & b255) + (lax.shift_right_logical(w, sh8) & b255)
                + (lax.shift_right_logical(w, sh16) & b255)
                + lax.shift_right_logical(w, sh24))
        m = (c < sid).astype(jnp.int32)
        return acc_pre + bsum * m, acc_tot + bsum

    acc_pre, acc_tot = lax.fori_loop(0, NUM_SUBCORES, pref_body, (zero, zero))
    pos_before = jnp.sum(acc_pre)
    p_total = jnp.sum(acc_tot)

    tgt_d.wait()
    iot = lax.iota(jnp.int32, 16)
    carry = jnp.int32(0)
    for k in range(VPC):
        t = tgt_v[pl.ds(k * 16, 16)]
        csum = plsc.cumsum(t)
        excl = csum - t
        prank = pos_before + carry + excl
        gidx = base + k * 16 + iot
        dest = jnp.where(t == 1, prank, M + gidx - prank)
        idx_v[k // 8, pl.ds((k % 8) * 16, 16)] = dest
        carry = carry + csum[15]

    for d in sc_d:
        d.wait()
    scat_d = [pltpu.async_copy(sc_v.at[q], cshared.at[idx_v.at[q]], sem2)
              for q in range(8)]
    for d in scat_d:
        d.wait()

    plsc.subcore_barrier()

    pltpu.sync_copy(cshared.at[pl.ds(sid * (2 * M // NUM_SUBCORES),
                                     2 * M // NUM_SUBCORES)],
                    bounce_hbm.at[pl.ds(sid * (2 * M // NUM_SUBCORES),
                                        2 * M // NUM_SUBCORES)])

    plsc.subcore_barrier()

    neg_copy = pltpu.async_copy(bounce_hbm.at[pl.ds(M, M)], negv, sem0)
    pos_copy = pltpu.async_copy(bounce_hbm.at[pl.ds(base, CHUNK)], posv, sem2)

    P = jnp.broadcast_to(p_total, (16,))
    s_i = jnp.maximum(M - P, 1)
    s_u = plsc.bitcast(s_i, jnp.uint32)
    s_f = s_i.astype(jnp.float32)
    rcp = 1.0 / s_f
    m1 = lax.rem(jnp.full((16,), 65536, jnp.uint32), s_u)
    m2 = lax.rem(m1 * m1, s_u)
    a3 = lax.rem(m2 * m1, s_u)
    off = s_i * (2 + lax.div(1024 + s_i - 1, s_i))
    lim = jnp.full((16,), 2.0e9, jnp.float32)
    big = jnp.full((16,), 4294967296.0, jnp.float32)
    mask16 = jnp.full((16,), 0xFFFF, jnp.uint32)
    is_one = s_i == 1

    bits_d.wait()
    pos_copy.wait()
    neg_copy.wait()

    def body(i, acc):
        sc = i // VPC
        k = i - sc * VPC
        hb = bitv[pl.ds(sc * 2 * CHUNK + k * 16, 16)]
        lb = bitv[pl.ds(sc * 2 * CHUNK + CHUNK + k * 16, 16)]
        h1 = lax.shift_right_logical(hb, jnp.uint32(16))
        h0 = hb & mask16
        l1 = lax.shift_right_logical(lb, jnp.uint32(16))
        l0 = lb & mask16
        v = h1 * a3 + h0 * m2 + l1 * m1 + l0
        vi = plsc.bitcast(v, jnp.int32)
        vf = vi.astype(jnp.float32)
        vf = jnp.where(vi < 0, vf + big, vf)
        q1 = jnp.minimum(vf * rcp, lim).astype(jnp.int32)
        r1u = v - plsc.bitcast(q1, jnp.uint32) * s_u
        r1 = plsc.bitcast(r1u, jnp.int32) + off
        q2 = (r1.astype(jnp.float32) * rcp).astype(jnp.int32)
        r2 = r1 - q2 * s_i
        r2 = jnp.where(r2 < 0, r2 + s_i, r2)
        r2 = jnp.where(r2 >= s_i, r2 - s_i, r2)
        samp = jnp.where(is_one, 0, r2)
        neg = plsc.load_gather(negv, [samp])
        p = posv[pl.ds(k * 16, 16)]
        rowid = base + k * 16 + iot
        term = jnp.maximum(MARGIN - p + neg, 0.0)
        return acc + jnp.where(rowid < P, term, 0.0)

    acc = lax.fori_loop(0, S * VPC, body, jnp.zeros((16,), jnp.float32))
    accv[...] = acc
    pltpu.sync_copy(accv, cshared.at[pl.ds(sid * 16, 16)])

    plsc.subcore_barrier()

    @pl.when(sid == 0)
    def _():
        pltpu.sync_copy(cshared.at[pl.ds(0, NUM_SUBCORES * 16)], redv)
        tot = jnp.zeros((16,), jnp.float32)
        for i in range(NUM_SUBCORES):
            tot = tot + redv[pl.ds(i * 16, 16)]
        total_v = jnp.broadcast_to(jnp.sum(tot), (16,))
        denom_v = (P * S).astype(jnp.float32)
        pv[...] = total_v / denom_v
        pltpu.sync_copy(pv, loss_hbm)


_fused = pl.kernel(
    _body,
    out_type=[jax.ShapeDtypeStruct((16,), jnp.float32),
              jax.ShapeDtypeStruct((2 * M,), jnp.float32)],
    mesh=_mesh,
    compiler_params=pltpu.CompilerParams(needs_layout_passes=False),
    scratch_types=[
        pltpu.VMEM((M // 4,), jnp.int32),
        pltpu.VMEM((CHUNK,), jnp.int32),
        pltpu.VMEM((8, 128), jnp.float32),
        pltpu.VMEM((8, 128), jnp.int32),
        pltpu.VMEM_SHARED((2 * M,), jnp.float32),
        pltpu.VMEM((M,), jnp.float32),
        pltpu.VMEM((CHUNK,), jnp.float32),
        pltpu.VMEM((BITS_PER_TILE,), jnp.uint32),
        pltpu.VMEM((NUM_SUBCORES * 16,), jnp.float32),
        pltpu.VMEM((16,), jnp.float32),
        pltpu.VMEM((16,), jnp.float32),
        pltpu.SemaphoreType.DMA,
        pltpu.SemaphoreType.DMA,
        pltpu.SemaphoreType.DMA,
    ],
)


def kernel(scores, target):
    bits = jnp.asarray(_BITS)
    tgt8 = lax.bitcast_convert_type(
        target.astype(jnp.int8).reshape(M // 4, 4), jnp.int32)
    loss, _ = _fused(scores, target, tgt8, bits)
    return loss[0]
